# baseline (speedup 1.0000x reference)
import functools
from itertools import product

import jax
import jax.numpy as jnp
import numpy as np
from jax.experimental import pallas as pl
from jax.experimental.pallas import tpu as pltpu

_B = 4096
_C = 6
_D = 512
_DOM_LEN = 64
_TMP_LEN = 720
_BS = 128
_NBUF = 4
_LOOK = 3


def _in_copies(s_hbm, t_hbm, s_buf, t_buf, insem, slot, step):
    cs = pltpu.make_async_copy(
        s_hbm.at[pl.ds(step * _BS, _BS)], s_buf.at[slot], insem.at[slot, 0])
    ct = pltpu.make_async_copy(
        t_hbm.at[pl.ds(step * _BS, _BS)], t_buf.at[slot], insem.at[slot, 1])
    return cs, ct


def _out_copies(s_buf, t_buf, out_hbm, outsem, slot, step):
    c1 = pltpu.make_async_copy(
        s_buf.at[slot], out_hbm.at[pl.ds(step * _BS, _BS)], outsem.at[slot, 0])
    c2 = pltpu.make_async_copy(
        t_buf.at[slot], out_hbm.at[pl.ds(_B + step * _BS, _BS)],
        outsem.at[slot, 1])
    return c1, c2


def _select_kernel(mask_ref, s_hbm, t_hbm, out_hbm,
                   s_buf, t_buf, insem, outsem):
    b = pl.program_id(0)
    nb = pl.num_programs(0)
    slot = jax.lax.rem(b, _NBUF)

    @pl.when(b == 0)
    def _():
        for k in range(_LOOK):
            cs, ct = _in_copies(s_hbm, t_hbm, s_buf, t_buf, insem, k, k)
            cs.start()
            ct.start()

    tgt = b + _LOOK

    @pl.when(tgt < nb)
    def _():
        tslot = jax.lax.rem(tgt, _NBUF)

        @pl.when(tgt >= _NBUF)
        def _():
            c1, c2 = _out_copies(s_buf, t_buf, out_hbm, outsem, tslot,
                                 tgt - _NBUF)
            c1.wait()
            c2.wait()

        cs, ct = _in_copies(s_hbm, t_hbm, s_buf, t_buf, insem, tslot, tgt)
        cs.start()
        ct.start()

    cs, ct = _in_copies(s_hbm, t_hbm, s_buf, t_buf, insem, slot, b)
    cs.wait()
    ct.wait()

    m = mask_ref[...]
    s = s_buf[slot]
    t = t_buf[slot]
    d = m * (t - s)
    s_buf[slot] = s + d
    t_buf[slot] = t - d

    c1, c2 = _out_copies(s_buf, t_buf, out_hbm, outsem, slot, b)
    c1.start()
    c2.start()

    @pl.when(b == nb - 1)
    def _():
        for k in range(_NBUF):
            step = nb - _NBUF + k
            c1, c2 = _out_copies(s_buf, t_buf, out_hbm, outsem,
                                 step % _NBUF, step)
            c1.wait()
            c2.wait()


@functools.partial(jax.jit, static_argnums=())
def _labels():
    rkey = jax.random.key(1234)
    ka, kb = jax.random.split(rkey)
    tem_rand_lab = jax.random.randint(ka, (_B,), 0, _TMP_LEN)
    dom_rand_lab1 = jax.random.randint(kb, (_B,), 0, _DOM_LEN // 2)
    return tem_rand_lab, dom_rand_lab1


def kernel(sfeat, tfeat):
    B, C, D = _B, _C, _D
    tem_rand_lab, dom_rand_lab1 = _labels()
    dom_set = jnp.asarray(
        np.array(list(product(*[[0, 1]] * C)), dtype=np.int32))
    bits = jnp.take(dom_set, dom_rand_lab1, axis=0)
    mask = bits.astype(jnp.float32)[:, :, None]

    nb = B // _BS
    feat = pl.pallas_call(
        _select_kernel,
        grid=(nb,),
        in_specs=[
            pl.BlockSpec((_BS, C, 1), lambda b: (b, 0, 0)),
            pl.BlockSpec(memory_space=pltpu.MemorySpace.HBM),
            pl.BlockSpec(memory_space=pltpu.MemorySpace.HBM),
        ],
        out_specs=pl.BlockSpec(memory_space=pltpu.MemorySpace.HBM),
        out_shape=jax.ShapeDtypeStruct((2 * B, C, D), sfeat.dtype),
        scratch_shapes=[
            pltpu.VMEM((_NBUF, _BS, C, D), jnp.float32),
            pltpu.VMEM((_NBUF, _BS, C, D), jnp.float32),
            pltpu.SemaphoreType.DMA((_NBUF, 2)),
            pltpu.SemaphoreType.DMA((_NBUF, 2)),
        ],
        compiler_params=pltpu.CompilerParams(
            dimension_semantics=("arbitrary",)),
    )(mask, sfeat, tfeat)

    dom_lab = jnp.concatenate([dom_rand_lab1, _DOM_LEN - 1 - dom_rand_lab1])
    tmp_lab = jnp.concatenate([tem_rand_lab, tem_rand_lab])
    dom_conf_lab = jnp.full((2 * B, _DOM_LEN), 1.0 / _DOM_LEN, jnp.float32)
    tmp_conf_lab = jnp.full((2 * B, _TMP_LEN), 1.0 / _TMP_LEN, jnp.float32)
    return (feat, dom_lab, dom_conf_lab, tmp_lab, tmp_conf_lab)
